# baseline (speedup 1.0000x reference)
import functools

import jax
import jax.numpy as jnp
from jax import lax
from jax.experimental import pallas as pl
from jax.experimental.pallas import tpu as pltpu
from jax.experimental.pallas import tpu_sc as plsc

EMBSIZE = 64
EPS = 1e-5

NUM_CORES = 2
NUM_SUBCORES = 16
NW = NUM_CORES * NUM_SUBCORES

CHUNK = 128
HIST = 200
BATCH = 4096


def _rsqrt(x):
    i = lax.bitcast_convert_type(x, jnp.int32)
    y = lax.bitcast_convert_type(jnp.int32(0x5F3759DF) - (i >> 1), jnp.float32)
    half = jnp.float32(0.5) * x
    for _ in range(3):
        y = y * (jnp.float32(1.5) - half * y * y)
    return y


def _make_kernel():
    mesh = plsc.VectorSubcoreMesh(core_axis_name="c", subcore_axis_name="s")

    @functools.partial(
        pl.kernel,
        out_type=jax.ShapeDtypeStruct((HIST, 8, NW, 8, CHUNK), jnp.float32),
        mesh=mesh,
        scratch_types=[
            pltpu.VMEM((HIST, CHUNK), jnp.int32),
            pltpu.VMEM((CHUNK, EMBSIZE), jnp.float32),
            pltpu.VMEM((CHUNK, EMBSIZE), jnp.float32),
            pltpu.VMEM((8, 8, CHUNK), jnp.float32),
            pltpu.VMEM((8, 8, CHUNK), jnp.float32),
            pltpu.VMEM((EMBSIZE,), jnp.float32),
            pltpu.VMEM((EMBSIZE,), jnp.float32),
            pltpu.SemaphoreType.DMA,
            pltpu.SemaphoreType.DMA,
            pltpu.SemaphoreType.DMA,
            pltpu.SemaphoreType.DMA,
        ],
        compiler_params=pltpu.CompilerParams(
            use_tc_tiling_on_sc=False, needs_layout_passes=False),
    )
    def k(idx_hbm, table_hbm, gamma_hbm, beta_hbm, out_hbm,
          idx_v, gbuf0, gbuf1, sbuf0, sbuf1, gamma_v, beta_v,
          g0, g1, s0, s1):
        wid = lax.axis_index("s") * NUM_CORES + lax.axis_index("c")
        gbufs = [gbuf0, gbuf1]
        sbufs = [sbuf0, sbuf1]
        gsem = [g0, g1]
        ssem = [s0, s1]
        pltpu.sync_copy(idx_hbm.at[:, wid], idx_v)
        pltpu.sync_copy(gamma_hbm, gamma_v)
        pltpu.sync_copy(beta_hbm, beta_v)

        inv_n = jnp.float32(1.0 / EMBSIZE)
        lane = lax.iota(jnp.int32, 16)

        def start_gather(c, t):
            pltpu.async_copy(table_hbm.at[idx_v.at[c]], gbufs[t], gsem[t])

        def wait_gather(c, t):
            pltpu.make_async_copy(
                table_hbm.at[idx_v.at[c]], gbufs[t], gsem[t]).wait()

        def out_slice(c):
            return out_hbm.at[c, :, wid]

        def start_store(c, t):
            pltpu.async_copy(sbufs[t], out_slice(c), ssem[t])

        def wait_store(c, t):
            pltpu.make_async_copy(sbufs[t], out_slice(c), ssem[t]).wait()

        def process(gbuf, sbuf):
            means = []
            istds = []
            for grp in range(8):
                rows = jnp.int32(grp * 16) + lane

                def p1(e, carry):
                    acc_s, acc_q = carry
                    col = jnp.full((16,), e, dtype=jnp.int32)
                    v = plsc.load_gather(gbuf, [rows, col])
                    sbuf[e >> 3, e & 7, pl.ds(grp * 16, 16)] = v
                    return acc_s + v, acc_q + v * v

                zero = jnp.zeros((16,), jnp.float32)
                acc_s, acc_q = lax.fori_loop(
                    0, EMBSIZE, p1, (zero, zero), unroll=8)
                mean = acc_s * inv_n
                var = acc_q * inv_n - mean * mean
                means.append(mean)
                istds.append(_rsqrt(var + jnp.float32(EPS)))

            def p2(f, carry):
                colf = jnp.full((16,), f, dtype=jnp.int32)
                gam = plsc.load_gather(gamma_v, [colf])
                bet = plsc.load_gather(beta_v, [colf])
                for grp in range(8):
                    v = sbuf[f >> 3, f & 7, pl.ds(grp * 16, 16)]
                    sbuf[f >> 3, f & 7, pl.ds(grp * 16, 16)] = (
                        (v - means[grp]) * istds[grp] * gam + bet)
                return carry

            lax.fori_loop(0, EMBSIZE, p2, 0, unroll=2)

        start_gather(0, 0)

        def pair_body(i, carry):
            for t in range(2):
                c = i * 2 + t
                tn = 1 - t
                @pl.when(c + 1 < HIST)
                def _():
                    start_gather(c + 1, tn)
                wait_gather(c, t)
                @pl.when(c >= 2)
                def _():
                    wait_store(c - 2, t)
                process(gbufs[t], sbufs[t])
                start_store(c, t)
            return carry

        lax.fori_loop(0, HIST // 2, pair_body, 0)
        wait_store(HIST - 2, 0)
        wait_store(HIST - 1, 1)

    return k


@jax.jit
def kernel(x, table, ln_gamma, ln_beta):
    idx = jnp.transpose(x).reshape(HIST, NW, CHUNK)
    out5 = _make_kernel()(idx, table, ln_gamma, ln_beta)
    return out5.transpose(2, 4, 0, 1, 3).reshape(BATCH, HIST, EMBSIZE)


# speedup vs baseline: 2.3482x; 2.3482x over previous
import functools

import jax
import jax.numpy as jnp
from jax import lax
from jax.experimental import pallas as pl
from jax.experimental.pallas import tpu as pltpu
from jax.experimental.pallas import tpu_sc as plsc

EMBSIZE = 64
EPS = 1e-5

NUM_CORES = 2
NUM_SUBCORES = 16
NW = NUM_CORES * NUM_SUBCORES

CHUNK = 128
HIST = 200
BATCH = 4096


def _rsqrt(x):
    i = lax.bitcast_convert_type(x, jnp.int32)
    y = lax.bitcast_convert_type(jnp.int32(0x5F3759DF) - (i >> 1), jnp.float32)
    half = jnp.float32(0.5) * x
    for _ in range(3):
        y = y * (jnp.float32(1.5) - half * y * y)
    return y


def _make_kernel():
    mesh = plsc.VectorSubcoreMesh(core_axis_name="c", subcore_axis_name="s")

    @functools.partial(
        pl.kernel,
        out_type=jax.ShapeDtypeStruct((HIST, 8, NW, 8, CHUNK), jnp.float32),
        mesh=mesh,
        scratch_types=[
            pltpu.VMEM((HIST, CHUNK), jnp.int32),
            pltpu.VMEM((CHUNK, 2 * EMBSIZE), jnp.float32),
            pltpu.VMEM((CHUNK, 2 * EMBSIZE), jnp.float32),
            pltpu.VMEM((EMBSIZE, CHUNK), jnp.float32),
            pltpu.VMEM((EMBSIZE, CHUNK), jnp.float32),
            pltpu.VMEM((EMBSIZE,), jnp.float32),
            pltpu.VMEM((EMBSIZE,), jnp.float32),
            pltpu.SemaphoreType.DMA,
            pltpu.SemaphoreType.DMA,
            pltpu.SemaphoreType.DMA,
            pltpu.SemaphoreType.DMA,
        ],
        compiler_params=pltpu.CompilerParams(
            use_tc_tiling_on_sc=False, needs_layout_passes=False),
    )
    def k(idx_hbm, table_hbm, gamma_hbm, beta_hbm, out_hbm,
          idx_v, gbuf0, gbuf1, sbuf0, sbuf1, gamma_v, beta_v,
          g0, g1, s0, s1):
        wid = lax.axis_index("s") * NUM_CORES + lax.axis_index("c")
        gbufs = [gbuf0, gbuf1]
        sbufs = [sbuf0, sbuf1]
        gsem = [g0, g1]
        ssem = [s0, s1]
        pltpu.sync_copy(idx_hbm.at[:, wid], idx_v)
        pltpu.sync_copy(gamma_hbm, gamma_v)
        pltpu.sync_copy(beta_hbm, beta_v)

        inv_n = jnp.float32(1.0 / EMBSIZE)
        lane = lax.iota(jnp.int32, 16)

        def start_gather(c, t):
            pltpu.async_copy(table_hbm.at[idx_v.at[c]], gbufs[t], gsem[t])

        def wait_gather(c, t):
            pltpu.make_async_copy(
                table_hbm.at[idx_v.at[c]], gbufs[t], gsem[t]).wait()

        def start_store(c, t):
            for et in range(8):
                pltpu.async_copy(sbufs[t].at[pl.ds(et * 8, 8)],
                                 out_hbm.at[c, et, wid], ssem[t])

        def wait_store(c, t):
            for et in range(8):
                pltpu.make_async_copy(sbufs[t].at[pl.ds(et * 8, 8)],
                                      out_hbm.at[c, et, wid], ssem[t]).wait()

        rows = [jnp.int32(grp * 16) + lane for grp in range(8)]
        c63 = jnp.int32(63)

        def process(gbuf, sbuf):
            def p1(e, carry):
                accs = list(carry[0])
                accq = list(carry[1])
                colv = carry[2]
                for grp in range(8):
                    v = plsc.load_gather(gbuf, [rows[grp], colv])
                    accs[grp] = accs[grp] + v
                    accq[grp] = accq[grp] + v * v
                return tuple(accs), tuple(accq), (colv + jnp.int32(1)) & c63

            zero = jnp.zeros((16,), jnp.float32)
            accs, accq, _unused = lax.fori_loop(
                0, EMBSIZE, p1, ((zero,) * 8, (zero,) * 8, lane), unroll=2)
            means = [a * inv_n for a in accs]
            istds = [_rsqrt(q * inv_n - m * m + jnp.float32(EPS))
                     for q, m in zip(accq, means)]

            @plsc.parallel_loop(0, EMBSIZE, unroll=2, carry=lane)
            def p2(e, colv):
                gam = plsc.load_gather(gamma_v, [colv])
                bet = plsc.load_gather(beta_v, [colv])
                for grp in range(8):
                    v = plsc.load_gather(gbuf, [rows[grp], colv])
                    w = (v - means[grp]) * istds[grp] * gam + bet
                    plsc.store_scatter(sbuf, [colv, rows[grp]], w)
                return (colv + jnp.int32(1)) & c63

        start_gather(0, 0)

        def pair_body(i, carry):
            for t in range(2):
                c = i * 2 + t
                tn = 1 - t
                @pl.when(c + 1 < HIST)
                def _():
                    start_gather(c + 1, tn)
                wait_gather(c, t)
                @pl.when(c >= 2)
                def _():
                    wait_store(c - 2, t)
                process(gbufs[t], sbufs[t])
                start_store(c, t)
            return carry

        lax.fori_loop(0, HIST // 2, pair_body, 0)
        wait_store(HIST - 2, 0)
        wait_store(HIST - 1, 1)

    return k


@jax.jit
def kernel(x, table, ln_gamma, ln_beta):
    tbl2 = jnp.pad(table, ((0, 0), (0, EMBSIZE)))
    idx = jnp.transpose(x).reshape(HIST, NW, CHUNK)
    out5 = _make_kernel()(idx, tbl2, ln_gamma, ln_beta)
    return out5.transpose(2, 4, 0, 1, 3).reshape(BATCH, HIST, EMBSIZE)
